# baseline (speedup 1.0000x reference)
import functools

import jax
import jax.numpy as jnp
from jax import lax
from jax.experimental import pallas as pl
from jax.experimental.pallas import tpu as pltpu
from jax.experimental.pallas import tpu_sc as plsc

T = 2048
H = 1024
FF = 4096
FF2 = FF // 2
E = 8
NE = 10
TOPK = 2
TM = 256
NB = (T * TOPK) // TM + E
NPAD = NB * TM

_F32 = jnp.float32
_BF16 = jnp.bfloat16

_NW = 32
_RPW = NPAD // _NW
_CHUNK = 64


def _sc_gather_body(x_hbm, idx_hbm, out_hbm, idx_v, rows_v, sem):
    wid = lax.axis_index("s") * 2 + lax.axis_index("c")
    base = wid * _RPW
    for ci in range(_RPW // _CHUNK):
        off = base + ci * _CHUNK
        pltpu.sync_copy(idx_hbm.at[pl.ds(off, _CHUNK)], idx_v)
        pltpu.async_copy(x_hbm.at[idx_v], rows_v, sem).wait()
        pltpu.sync_copy(rows_v, out_hbm.at[pl.ds(off, _CHUNK)])


def _sc_gather(x, row_tok):
    k = functools.partial(
        pl.kernel,
        mesh=plsc.VectorSubcoreMesh(core_axis_name="c",
                                    subcore_axis_name="s"),
        out_type=jax.ShapeDtypeStruct((NPAD, H), _F32),
        scratch_types=[
            pltpu.VMEM((_CHUNK,), jnp.int32),
            pltpu.VMEM((_CHUNK, H), _F32),
            pltpu.SemaphoreType.DMA,
        ],
    )(_sc_gather_body)
    return k(x, row_tok)


def _router_body(x_ref, g_ref, logits_ref, meta_ref):
    x = x_ref[...]
    g = g_ref[...]
    logits = lax.dot_general(x, g, (((1,), (1,)), ((), ())),
                             preferred_element_type=_F32)
    logits_ref[...] = logits
    lane = lax.broadcasted_iota(jnp.int32, (T, 16), 1)
    masked = jnp.where(lane < NE, logits, -1e30)
    m = jnp.max(masked, axis=1, keepdims=True)
    ex = jnp.exp(masked - m)
    p = ex / jnp.sum(ex, axis=1, keepdims=True)
    p1 = jnp.max(p, axis=1, keepdims=True)
    i1 = jnp.min(jnp.where(p == p1, lane, 999), axis=1, keepdims=True)
    pm = jnp.where(lane == i1, -1.0, p)
    p2 = jnp.max(pm, axis=1, keepdims=True)
    i2 = jnp.min(jnp.where(pm == p2, lane, 999), axis=1, keepdims=True)
    m1 = (i1 < E).astype(_F32)
    m2 = (i2 < E).astype(_F32)
    s = p1 * m1 + p2 * m2
    d = jnp.where(s == 0.0, 1.0, s)
    w1 = p1 * m1 / d
    w2 = p2 * m2 / d
    lane8 = lax.broadcasted_iota(jnp.int32, (T, 8), 1)
    meta = jnp.where(lane8 == 0, w1,
                     jnp.where(lane8 == 1, w2,
                               jnp.where(lane8 == 2, i1.astype(_F32),
                                         i2.astype(_F32))))
    meta_ref[...] = meta


def _h1_body(be_ref, na_ref, xs_ref, w1_ref, w3_ref, h1_ref):
    b = pl.program_id(0)
    active = b < na_ref[0]

    @pl.when(active)
    def _compute():
        xt = xs_ref[0]
        a = lax.dot_general(xt, w1_ref[0], (((1,), (1,)), ((), ())),
                            preferred_element_type=_F32)
        c = lax.dot_general(xt, w3_ref[0], (((1,), (1,)), ((), ())),
                            preferred_element_type=_F32)
        h1_ref[0] = ((a * jax.nn.sigmoid(a)) * c).astype(_BF16)


def _out_body(be_ref, na_ref, tok_ref, ww_ref, h1a_ref, h1b_ref, w2_ref,
              out_ref, w2b_scr):
    b = pl.program_id(0)
    active = b < na_ref[0]

    @pl.when(b == 0)
    def _init():
        out_ref[...] = jnp.zeros_like(out_ref)

    new_w = (b == 0) | (be_ref[b] != be_ref[jnp.maximum(b - 1, 0)])

    @pl.when(active & new_w)
    def _cast():
        w2b_scr[...] = w2_ref[0].astype(_BF16)

    @pl.when(active)
    def _compute():
        oc = lax.dot_general(h1a_ref[0], w2b_scr[:, :FF2],
                             (((1,), (1,)), ((), ())),
                             preferred_element_type=_F32)
        oc = oc + lax.dot_general(h1b_ref[0], w2b_scr[:, FF2:],
                                  (((1,), (1,)), ((), ())),
                                  preferred_element_type=_F32)
        ww = ww_ref[0, 0, :]
        ow = oc * ww[:, None]
        ids = tok_ref[0, 0, :]
        cols = lax.broadcasted_iota(jnp.int32, (TM, T), 1)
        gb = (ids[:, None] == cols).astype(_F32)
        out_ref[...] += lax.dot_general(gb, ow, (((0,), (0,)), ((), ())),
                                        preferred_element_type=_F32)


def _h1_call(half, be, nact_arr, xs3, W1, W3):
    grid_spec = pltpu.PrefetchScalarGridSpec(
        num_scalar_prefetch=2,
        grid=(NB,),
        in_specs=[
            pl.BlockSpec((1, TM, H), lambda b, be, na: (b, 0, 0)),
            pl.BlockSpec((1, FF2, H),
                         lambda b, be, na: (be[b], half, 0)),
            pl.BlockSpec((1, FF2, H),
                         lambda b, be, na: (be[b], half, 0)),
        ],
        out_specs=pl.BlockSpec((1, TM, FF2), lambda b, be, na: (b, 0, 0)),
    )
    return pl.pallas_call(
        _h1_body,
        grid_spec=grid_spec,
        out_shape=jax.ShapeDtypeStruct((NB, TM, FF2), _BF16),
        compiler_params=pltpu.CompilerParams(
            dimension_semantics=("arbitrary",),
            vmem_limit_bytes=60 * 1024 * 1024),
    )(be, nact_arr, xs3, W1, W3)


def kernel(hidden_states, gate_w, gate2_w, W1, W2, W3):
    b, s, h = hidden_states.shape
    x = hidden_states.reshape(T, H)

    gates = jnp.concatenate(
        [gate_w, gate2_w, jnp.zeros((16 - NE, H), _F32)], axis=0)

    logits16, meta = pl.pallas_call(
        _router_body,
        out_shape=(jax.ShapeDtypeStruct((T, 16), _F32),
                   jax.ShapeDtypeStruct((T, 8), _F32)),
    )(x, gates)

    router_logits = logits16[:, :NE]

    wts = meta[:, 0:2]
    eids = meta[:, 2:4].astype(jnp.int32)
    e_flat = eids.reshape(-1)
    w_flat = wts.reshape(-1)
    tok = jnp.arange(T * TOPK, dtype=jnp.int32) // TOPK
    key = jnp.where(e_flat < E, e_flat, E)
    cnt = jnp.bincount(key, length=E + 1)[:E].astype(jnp.int32)
    blocks_per = (cnt + TM - 1) // TM
    blk_start = jnp.concatenate(
        [jnp.zeros((1,), jnp.int32), jnp.cumsum(blocks_per)[:-1]])
    nact = jnp.sum(blocks_per).astype(jnp.int32)
    run_start = jnp.concatenate(
        [jnp.zeros((1,), jnp.int32), jnp.cumsum(cnt)])
    perm = jnp.argsort(key, stable=True)
    se = key[perm]
    st = tok[perm]
    sw = w_flat[perm]
    pos_in_run = jnp.arange(T * TOPK, dtype=jnp.int32) - run_start[se]
    dest = jnp.where(se < E,
                     TM * blk_start[jnp.minimum(se, E - 1)] + pos_in_run,
                     NPAD)
    row_tok = jnp.zeros((NPAD + 1,), jnp.int32).at[dest].set(st)[:NPAD]
    row_w = jnp.zeros((NPAD + 1,), _F32).at[dest].set(sw)[:NPAD]
    bidx = jnp.arange(NB, dtype=jnp.int32)
    be = jnp.sum(bidx[:, None] >= blk_start[None, :], axis=1).astype(jnp.int32) - 1
    be_last = be[jnp.maximum(nact - 1, 0)]
    be = jnp.where(bidx < nact, be, be_last)
    nact_arr = nact.reshape(1)

    tok3 = row_tok.reshape(NB, 1, TM)
    ww3 = row_w.reshape(NB, 1, TM)

    xs = _sc_gather(x, row_tok)
    xs3 = xs.reshape(NB, TM, H)

    h1a = _h1_call(0, be, nact_arr, xs3, W1, W3)
    h1b = _h1_call(1, be, nact_arr, xs3, W1, W3)

    grid_spec = pltpu.PrefetchScalarGridSpec(
        num_scalar_prefetch=2,
        grid=(NB,),
        in_specs=[
            pl.BlockSpec((1, 1, TM), lambda b, be, na: (b, 0, 0)),
            pl.BlockSpec((1, 1, TM), lambda b, be, na: (b, 0, 0)),
            pl.BlockSpec((1, TM, FF2), lambda b, be, na: (b, 0, 0)),
            pl.BlockSpec((1, TM, FF2), lambda b, be, na: (b, 0, 0)),
            pl.BlockSpec((1, H, FF), lambda b, be, na: (be[b], 0, 0)),
        ],
        out_specs=pl.BlockSpec((T, H), lambda b, be, na: (0, 0)),
        scratch_shapes=[
            pltpu.VMEM((H, FF), _BF16),
        ],
    )
    final = pl.pallas_call(
        _out_body,
        grid_spec=grid_spec,
        out_shape=jax.ShapeDtypeStruct((T, H), _F32),
        compiler_params=pltpu.CompilerParams(
            dimension_semantics=("arbitrary",),
            vmem_limit_bytes=60 * 1024 * 1024),
    )(be, nact_arr, tok3, ww3, h1a, h1b, W2)

    return final.reshape(b, s, h), router_logits


# speedup vs baseline: 1.3277x; 1.3277x over previous
import jax
import jax.numpy as jnp
from jax import lax
from jax.experimental import pallas as pl
from jax.experimental.pallas import tpu as pltpu

T = 2048
H = 1024
FF = 4096
FF2 = FF // 2
E = 8
NE = 10
TOPK = 2
TM = 256
NB = (T * TOPK) // TM + E
NPAD = NB * TM

_F32 = jnp.float32
_BF16 = jnp.bfloat16


def _router_body(x_ref, g_ref, logits_ref, meta_ref):
    x = x_ref[...]
    g = g_ref[...]
    logits = lax.dot_general(x, g, (((1,), (1,)), ((), ())),
                             preferred_element_type=_F32)
    logits_ref[...] = logits
    lane = lax.broadcasted_iota(jnp.int32, (T, 16), 1)
    masked = jnp.where(lane < NE, logits, -1e30)
    m = jnp.max(masked, axis=1, keepdims=True)
    ex = jnp.exp(masked - m)
    p = ex / jnp.sum(ex, axis=1, keepdims=True)
    p1 = jnp.max(p, axis=1, keepdims=True)
    i1 = jnp.min(jnp.where(p == p1, lane, 999), axis=1, keepdims=True)
    pm = jnp.where(lane == i1, -1.0, p)
    p2 = jnp.max(pm, axis=1, keepdims=True)
    i2 = jnp.min(jnp.where(pm == p2, lane, 999), axis=1, keepdims=True)
    m1 = (i1 < E).astype(_F32)
    m2 = (i2 < E).astype(_F32)
    s = p1 * m1 + p2 * m2
    d = jnp.where(s == 0.0, 1.0, s)
    w1 = p1 * m1 / d
    w2 = p2 * m2 / d
    lane8 = lax.broadcasted_iota(jnp.int32, (T, 8), 1)
    meta = jnp.where(lane8 == 0, w1,
                     jnp.where(lane8 == 1, w2,
                               jnp.where(lane8 == 2, i1.astype(_F32),
                                         i2.astype(_F32))))
    meta_ref[...] = meta


def _h1_body(be_ref, na_ref, tok_ref, x_ref, w1_ref, w3_ref, h1_ref):
    b = pl.program_id(1)
    active = b < na_ref[0]

    @pl.when(active)
    def _compute():
        ids = tok_ref[0, 0, :]
        cols = lax.broadcasted_iota(jnp.int32, (TM, T), 1)
        gb = (ids[:, None] == cols).astype(_F32)
        xt = jnp.dot(gb, x_ref[...], preferred_element_type=_F32)
        a = lax.dot_general(xt, w1_ref[0], (((1,), (1,)), ((), ())),
                            preferred_element_type=_F32)
        c = lax.dot_general(xt, w3_ref[0], (((1,), (1,)), ((), ())),
                            preferred_element_type=_F32)
        h1_ref[0, 0] = ((a * jax.nn.sigmoid(a)) * c).astype(_BF16)


def _out_body(be_ref, na_ref, tok_ref, ww_ref, h1a_ref, h1b_ref, w2_ref,
              out_ref, w2b_scr):
    b = pl.program_id(0)
    active = b < na_ref[0]

    @pl.when(b == 0)
    def _init():
        out_ref[...] = jnp.zeros_like(out_ref)

    new_w = (b == 0) | (be_ref[b] != be_ref[jnp.maximum(b - 1, 0)])

    @pl.when(active & new_w)
    def _cast():
        w2b_scr[...] = w2_ref[0].astype(_BF16)

    @pl.when(active)
    def _compute():
        oc = lax.dot_general(h1a_ref[0, 0], w2b_scr[:, :FF2],
                             (((1,), (1,)), ((), ())),
                             preferred_element_type=_F32)
        oc = oc + lax.dot_general(h1b_ref[0, 0], w2b_scr[:, FF2:],
                                  (((1,), (1,)), ((), ())),
                                  preferred_element_type=_F32)
        ww = ww_ref[0, 0, :]
        ow = oc * ww[:, None]
        ids = tok_ref[0, 0, :]
        cols = lax.broadcasted_iota(jnp.int32, (TM, T), 1)
        gb = (ids[:, None] == cols).astype(_F32)
        out_ref[...] += lax.dot_general(gb, ow, (((0,), (0,)), ((), ())),
                                        preferred_element_type=_F32)


def _h1_call(be, nact_arr, tok3, x, W1, W3):
    grid_spec = pltpu.PrefetchScalarGridSpec(
        num_scalar_prefetch=2,
        grid=(2, NB),
        in_specs=[
            pl.BlockSpec((1, 1, TM), lambda hf, b, be, na: (b, 0, 0)),
            pl.BlockSpec((T, H), lambda hf, b, be, na: (0, 0)),
            pl.BlockSpec((1, FF2, H),
                         lambda hf, b, be, na: (be[b], hf, 0)),
            pl.BlockSpec((1, FF2, H),
                         lambda hf, b, be, na: (be[b], hf, 0)),
        ],
        out_specs=pl.BlockSpec((1, 1, TM, FF2),
                               lambda hf, b, be, na: (hf, b, 0, 0)),
    )
    return pl.pallas_call(
        _h1_body,
        grid_spec=grid_spec,
        out_shape=jax.ShapeDtypeStruct((2, NB, TM, FF2), _BF16),
        compiler_params=pltpu.CompilerParams(
            dimension_semantics=("arbitrary", "arbitrary"),
            vmem_limit_bytes=60 * 1024 * 1024),
    )(be, nact_arr, tok3, x, W1, W3)


def kernel(hidden_states, gate_w, gate2_w, W1, W2, W3):
    b, s, h = hidden_states.shape
    x = hidden_states.reshape(T, H)

    gates = jnp.concatenate(
        [gate_w, gate2_w, jnp.zeros((16 - NE, H), _F32)], axis=0)

    logits16, meta = pl.pallas_call(
        _router_body,
        out_shape=(jax.ShapeDtypeStruct((T, 16), _F32),
                   jax.ShapeDtypeStruct((T, 8), _F32)),
    )(x, gates)

    router_logits = logits16[:, :NE]

    wts = meta[:, 0:2]
    eids = meta[:, 2:4].astype(jnp.int32)
    e_flat = eids.reshape(-1)
    w_flat = wts.reshape(-1)
    tok = jnp.arange(T * TOPK, dtype=jnp.int32) // TOPK
    key = jnp.where(e_flat < E, e_flat, E)
    cnt = jnp.bincount(key, length=E + 1)[:E].astype(jnp.int32)
    blocks_per = (cnt + TM - 1) // TM
    blk_start = jnp.concatenate(
        [jnp.zeros((1,), jnp.int32), jnp.cumsum(blocks_per)[:-1]])
    nact = jnp.sum(blocks_per).astype(jnp.int32)
    run_start = jnp.concatenate(
        [jnp.zeros((1,), jnp.int32), jnp.cumsum(cnt)])
    perm = jnp.argsort(key, stable=True)
    se = key[perm]
    st = tok[perm]
    sw = w_flat[perm]
    pos_in_run = jnp.arange(T * TOPK, dtype=jnp.int32) - run_start[se]
    dest = jnp.where(se < E,
                     TM * blk_start[jnp.minimum(se, E - 1)] + pos_in_run,
                     NPAD)
    row_tok = jnp.zeros((NPAD + 1,), jnp.int32).at[dest].set(st)[:NPAD]
    row_w = jnp.zeros((NPAD + 1,), _F32).at[dest].set(sw)[:NPAD]
    bidx = jnp.arange(NB, dtype=jnp.int32)
    be = jnp.sum(bidx[:, None] >= blk_start[None, :], axis=1).astype(jnp.int32) - 1
    be_last = be[jnp.maximum(nact - 1, 0)]
    be = jnp.where(bidx < nact, be, be_last)
    nact_arr = nact.reshape(1)

    tok3 = row_tok.reshape(NB, 1, TM)
    ww3 = row_w.reshape(NB, 1, TM)

    h1 = _h1_call(be, nact_arr, tok3, x, W1, W3)

    grid_spec = pltpu.PrefetchScalarGridSpec(
        num_scalar_prefetch=2,
        grid=(NB,),
        in_specs=[
            pl.BlockSpec((1, 1, TM), lambda b, be, na: (b, 0, 0)),
            pl.BlockSpec((1, 1, TM), lambda b, be, na: (b, 0, 0)),
            pl.BlockSpec((1, 1, TM, FF2), lambda b, be, na: (0, b, 0, 0)),
            pl.BlockSpec((1, 1, TM, FF2), lambda b, be, na: (1, b, 0, 0)),
            pl.BlockSpec((1, H, FF), lambda b, be, na: (be[b], 0, 0)),
        ],
        out_specs=pl.BlockSpec((T, H), lambda b, be, na: (0, 0)),
        scratch_shapes=[
            pltpu.VMEM((H, FF), _BF16),
        ],
    )
    final = pl.pallas_call(
        _out_body,
        grid_spec=grid_spec,
        out_shape=jax.ShapeDtypeStruct((T, H), _F32),
        compiler_params=pltpu.CompilerParams(
            dimension_semantics=("arbitrary",),
            vmem_limit_bytes=60 * 1024 * 1024),
    )(be, nact_arr, tok3, ww3, h1, h1, W2)

    return final.reshape(b, s, h), router_logits


# speedup vs baseline: 1.4784x; 1.1135x over previous
import jax
import jax.numpy as jnp
from jax import lax
from jax.experimental import pallas as pl
from jax.experimental.pallas import tpu as pltpu

T = 2048
H = 1024
FF = 4096
FF2 = FF // 2
E = 8
NE = 10
TOPK = 2
TM = 256
NB = (T * TOPK) // TM + E
NPAD = NB * TM

_F32 = jnp.float32
_BF16 = jnp.bfloat16


def _router_body(x_ref, g_ref, logits_ref, meta_ref):
    x = x_ref[...]
    g = g_ref[...]
    logits = lax.dot_general(x, g, (((1,), (1,)), ((), ())),
                             preferred_element_type=_F32)
    logits_ref[...] = logits
    lane = lax.broadcasted_iota(jnp.int32, (T, 16), 1)
    masked = jnp.where(lane < NE, logits, -1e30)
    m = jnp.max(masked, axis=1, keepdims=True)
    ex = jnp.exp(masked - m)
    p = ex / jnp.sum(ex, axis=1, keepdims=True)
    p1 = jnp.max(p, axis=1, keepdims=True)
    i1 = jnp.min(jnp.where(p == p1, lane, 999), axis=1, keepdims=True)
    pm = jnp.where(lane == i1, -1.0, p)
    p2 = jnp.max(pm, axis=1, keepdims=True)
    i2 = jnp.min(jnp.where(pm == p2, lane, 999), axis=1, keepdims=True)
    m1 = (i1 < E).astype(_F32)
    m2 = (i2 < E).astype(_F32)
    s = p1 * m1 + p2 * m2
    d = jnp.where(s == 0.0, 1.0, s)
    w1 = p1 * m1 / d
    w2 = p2 * m2 / d
    lane8 = lax.broadcasted_iota(jnp.int32, (T, 8), 1)
    meta = jnp.where(lane8 == 0, w1,
                     jnp.where(lane8 == 1, w2,
                               jnp.where(lane8 == 2, i1.astype(_F32),
                                         i2.astype(_F32))))
    meta_ref[...] = meta


def _h1_body(be_ref, na_ref, tok_ref, x_ref, w1_ref, w3_ref, h1_ref):
    b = pl.program_id(1)
    active = b < na_ref[0]

    @pl.when(active)
    def _compute():
        ids = tok_ref[0, 0, :]
        cols = lax.broadcasted_iota(jnp.int32, (TM, T), 1)
        gb = (ids[:, None] == cols).astype(_F32)
        xt = jnp.dot(gb, x_ref[...], preferred_element_type=_F32)
        a = lax.dot_general(xt, w1_ref[0], (((1,), (1,)), ((), ())),
                            preferred_element_type=_F32)
        c = lax.dot_general(xt, w3_ref[0], (((1,), (1,)), ((), ())),
                            preferred_element_type=_F32)
        h1_ref[0, 0] = ((a * jax.nn.sigmoid(a)) * c).astype(_BF16)


def _out_body(be_ref, na_ref, tok_ref, ww_ref, h1a_ref, h1b_ref, w2_ref,
              out_ref, w2b_scr):
    b = pl.program_id(0)
    active = b < na_ref[0]

    @pl.when(b == 0)
    def _init():
        out_ref[...] = jnp.zeros_like(out_ref)

    new_w = (b == 0) | (be_ref[b] != be_ref[jnp.maximum(b - 1, 0)])

    @pl.when(active & new_w)
    def _cast():
        w2b_scr[...] = w2_ref[0].astype(_BF16)

    @pl.when(active)
    def _compute():
        oc = lax.dot_general(h1a_ref[0, 0], w2b_scr[:, :FF2],
                             (((1,), (1,)), ((), ())),
                             preferred_element_type=_F32)
        oc = oc + lax.dot_general(h1b_ref[0, 0], w2b_scr[:, FF2:],
                                  (((1,), (1,)), ((), ())),
                                  preferred_element_type=_F32)
        ww = ww_ref[0, 0, :]
        ow = oc * ww[:, None]
        ids = tok_ref[0, 0, :]
        cols = lax.broadcasted_iota(jnp.int32, (TM, T), 1)
        gb = (ids[:, None] == cols).astype(_F32)
        out_ref[...] += lax.dot_general(gb, ow, (((0,), (0,)), ((), ())),
                                        preferred_element_type=_F32)


def _h1_call(be, nact_arr, tok3, x, W1, W3):
    grid_spec = pltpu.PrefetchScalarGridSpec(
        num_scalar_prefetch=2,
        grid=(2, NB),
        in_specs=[
            pl.BlockSpec((1, 1, TM), lambda hf, b, be, na: (b, 0, 0)),
            pl.BlockSpec((T, H), lambda hf, b, be, na: (0, 0)),
            pl.BlockSpec((1, FF2, H),
                         lambda hf, b, be, na: (be[b], hf, 0)),
            pl.BlockSpec((1, FF2, H),
                         lambda hf, b, be, na: (be[b], hf, 0)),
        ],
        out_specs=pl.BlockSpec((1, 1, TM, FF2),
                               lambda hf, b, be, na: (hf, b, 0, 0)),
    )
    return pl.pallas_call(
        _h1_body,
        grid_spec=grid_spec,
        out_shape=jax.ShapeDtypeStruct((2, NB, TM, FF2), _BF16),
        compiler_params=pltpu.CompilerParams(
            dimension_semantics=("arbitrary", "arbitrary"),
            vmem_limit_bytes=60 * 1024 * 1024),
    )(be, nact_arr, tok3, x, W1, W3)


def kernel(hidden_states, gate_w, gate2_w, W1, W2, W3):
    b, s, h = hidden_states.shape
    x = hidden_states.reshape(T, H)

    gates = jnp.concatenate(
        [gate_w, gate2_w, jnp.zeros((16 - NE, H), _F32)], axis=0)

    logits16, meta = pl.pallas_call(
        _router_body,
        out_shape=(jax.ShapeDtypeStruct((T, 16), _F32),
                   jax.ShapeDtypeStruct((T, 8), _F32)),
    )(x, gates)

    router_logits = logits16[:, :NE]

    wts = meta[:, 0:2]
    eids = meta[:, 2:4].astype(jnp.int32)
    e_flat = eids.reshape(-1)
    w_flat = wts.reshape(-1)
    tok = jnp.arange(T * TOPK, dtype=jnp.int32) // TOPK
    key = jnp.where(e_flat < E, e_flat, E)
    oh = (key[:, None] == jnp.arange(E + 1, dtype=jnp.int32)[None, :]
          ).astype(jnp.int32)
    cnt = jnp.sum(oh[:, :E], axis=0).astype(jnp.int32)
    blocks_per = (cnt + TM - 1) // TM
    blk_start = jnp.concatenate(
        [jnp.zeros((1,), jnp.int32), jnp.cumsum(blocks_per)[:-1]])
    nact = jnp.sum(blocks_per).astype(jnp.int32)
    rank = jnp.sum((jnp.cumsum(oh, axis=0) - oh) * oh, axis=1)
    dest = jnp.where(key < E,
                     TM * blk_start[jnp.minimum(key, E - 1)] + rank,
                     NPAD)
    row_tok = jnp.zeros((NPAD + 1,), jnp.int32).at[dest].set(tok)[:NPAD]
    row_w = jnp.zeros((NPAD + 1,), _F32).at[dest].set(w_flat)[:NPAD]
    bidx = jnp.arange(NB, dtype=jnp.int32)
    be = jnp.sum(bidx[:, None] >= blk_start[None, :], axis=1).astype(jnp.int32) - 1
    be_last = be[jnp.maximum(nact - 1, 0)]
    be = jnp.where(bidx < nact, be, be_last)
    nact_arr = nact.reshape(1)

    tok3 = row_tok.reshape(NB, 1, TM)
    ww3 = row_w.reshape(NB, 1, TM)

    h1 = _h1_call(be, nact_arr, tok3, x, W1, W3)

    grid_spec = pltpu.PrefetchScalarGridSpec(
        num_scalar_prefetch=2,
        grid=(NB,),
        in_specs=[
            pl.BlockSpec((1, 1, TM), lambda b, be, na: (b, 0, 0)),
            pl.BlockSpec((1, 1, TM), lambda b, be, na: (b, 0, 0)),
            pl.BlockSpec((1, 1, TM, FF2), lambda b, be, na: (0, b, 0, 0)),
            pl.BlockSpec((1, 1, TM, FF2), lambda b, be, na: (1, b, 0, 0)),
            pl.BlockSpec((1, H, FF), lambda b, be, na: (be[b], 0, 0)),
        ],
        out_specs=pl.BlockSpec((T, H), lambda b, be, na: (0, 0)),
        scratch_shapes=[
            pltpu.VMEM((H, FF), _BF16),
        ],
    )
    final = pl.pallas_call(
        _out_body,
        grid_spec=grid_spec,
        out_shape=jax.ShapeDtypeStruct((T, H), _F32),
        compiler_params=pltpu.CompilerParams(
            dimension_semantics=("arbitrary",),
            vmem_limit_bytes=60 * 1024 * 1024),
    )(be, nact_arr, tok3, ww3, h1, h1, W2)

    return final.reshape(b, s, h), router_logits


# speedup vs baseline: 1.4811x; 1.0018x over previous
import jax
import jax.numpy as jnp
from jax import lax
from jax.experimental import pallas as pl
from jax.experimental.pallas import tpu as pltpu

T = 2048
H = 1024
FF = 4096
FF2 = FF // 2
E = 8
NE = 10
TOPK = 2
TM = 256
NB = (T * TOPK) // TM + E
NPAD = NB * TM

_F32 = jnp.float32
_BF16 = jnp.bfloat16


def _router_body(x_ref, g_ref, logits_ref, meta_ref):
    x = x_ref[...]
    g = g_ref[...]
    logits = lax.dot_general(x, g, (((1,), (1,)), ((), ())),
                             preferred_element_type=_F32)
    logits_ref[...] = logits
    lane = lax.broadcasted_iota(jnp.int32, (T, 16), 1)
    masked = jnp.where(lane < NE, logits, -1e30)
    m = jnp.max(masked, axis=1, keepdims=True)
    ex = jnp.exp(masked - m)
    p = ex / jnp.sum(ex, axis=1, keepdims=True)
    p1 = jnp.max(p, axis=1, keepdims=True)
    i1 = jnp.min(jnp.where(p == p1, lane, 999), axis=1, keepdims=True)
    pm = jnp.where(lane == i1, -1.0, p)
    p2 = jnp.max(pm, axis=1, keepdims=True)
    i2 = jnp.min(jnp.where(pm == p2, lane, 999), axis=1, keepdims=True)
    m1 = (i1 < E).astype(_F32)
    m2 = (i2 < E).astype(_F32)
    s = p1 * m1 + p2 * m2
    d = jnp.where(s == 0.0, 1.0, s)
    w1 = p1 * m1 / d
    w2 = p2 * m2 / d
    lane8 = lax.broadcasted_iota(jnp.int32, (T, 8), 1)
    meta = jnp.where(lane8 == 0, w1,
                     jnp.where(lane8 == 1, w2,
                               jnp.where(lane8 == 2, i1.astype(_F32),
                                         i2.astype(_F32))))
    meta_ref[...] = meta


def _h1_body(be_ref, na_ref, tok_ref, x_ref, w1_ref, w3_ref, h1_ref):
    b = pl.program_id(1)
    active = b < na_ref[0]

    @pl.when(active)
    def _compute():
        ids = tok_ref[0, 0, :]
        cols = lax.broadcasted_iota(jnp.int32, (TM, T), 1)
        gb = (ids[:, None] == cols).astype(_F32)
        xt = jnp.dot(gb, x_ref[...], preferred_element_type=_F32)
        a = lax.dot_general(xt, w1_ref[0], (((1,), (1,)), ((), ())),
                            preferred_element_type=_F32)
        c = lax.dot_general(xt, w3_ref[0], (((1,), (1,)), ((), ())),
                            preferred_element_type=_F32)
        h1_ref[0, 0] = ((a * jax.nn.sigmoid(a)) * c).astype(_BF16)


def _out_body(be_ref, na_ref, tok_ref, ww_ref, h1a_ref, h1b_ref, w2_ref,
              out_ref, w2b_scr):
    b = pl.program_id(0)
    active = b < na_ref[0]

    @pl.when(b == 0)
    def _init():
        out_ref[...] = jnp.zeros_like(out_ref)

    new_w = (b == 0) | (be_ref[b] != be_ref[jnp.maximum(b - 1, 0)])

    @pl.when(active & new_w)
    def _cast():
        w2b_scr[...] = w2_ref[0].astype(_BF16)

    @pl.when(active)
    def _compute():
        oc = lax.dot_general(h1a_ref[0, 0], w2b_scr[:, :FF2],
                             (((1,), (1,)), ((), ())),
                             preferred_element_type=_F32)
        oc = oc + lax.dot_general(h1b_ref[0, 0], w2b_scr[:, FF2:],
                                  (((1,), (1,)), ((), ())),
                                  preferred_element_type=_F32)
        ww = ww_ref[0, 0, :]
        ow = oc * ww[:, None]
        ids = tok_ref[0, 0, :]
        cols = lax.broadcasted_iota(jnp.int32, (TM, T), 1)
        gb = (ids[:, None] == cols).astype(_F32)
        out_ref[...] += lax.dot_general(gb, ow, (((0,), (0,)), ((), ())),
                                        preferred_element_type=_F32)


def _h1_call(be, nact_arr, tok3, x, W1, W3):
    grid_spec = pltpu.PrefetchScalarGridSpec(
        num_scalar_prefetch=2,
        grid=(2, NB),
        in_specs=[
            pl.BlockSpec((1, 1, TM), lambda hf, b, be, na: (b, 0, 0)),
            pl.BlockSpec((T, H), lambda hf, b, be, na: (0, 0)),
            pl.BlockSpec((1, FF2, H),
                         lambda hf, b, be, na: (be[b], hf, 0)),
            pl.BlockSpec((1, FF2, H),
                         lambda hf, b, be, na: (be[b], hf, 0)),
        ],
        out_specs=pl.BlockSpec((1, 1, TM, FF2),
                               lambda hf, b, be, na: (hf, b, 0, 0)),
    )
    return pl.pallas_call(
        _h1_body,
        grid_spec=grid_spec,
        out_shape=jax.ShapeDtypeStruct((2, NB, TM, FF2), _BF16),
        compiler_params=pltpu.CompilerParams(
            dimension_semantics=("arbitrary", "arbitrary"),
            vmem_limit_bytes=60 * 1024 * 1024),
    )(be, nact_arr, tok3, x, W1, W3)


def kernel(hidden_states, gate_w, gate2_w, W1, W2, W3):
    b, s, h = hidden_states.shape
    x = hidden_states.reshape(T, H)

    gates = jnp.concatenate(
        [gate_w, gate2_w, jnp.zeros((16 - NE, H), _F32)], axis=0)

    logits16, meta = pl.pallas_call(
        _router_body,
        out_shape=(jax.ShapeDtypeStruct((T, 16), _F32),
                   jax.ShapeDtypeStruct((T, 8), _F32)),
    )(x, gates)

    router_logits = logits16[:, :NE]

    wts = meta[:, 0:2]
    eids = meta[:, 2:4].astype(jnp.int32)
    e_flat = eids.reshape(-1)
    w_flat = wts.reshape(-1)
    tok = jnp.arange(T * TOPK, dtype=jnp.int32) // TOPK
    key = jnp.where(e_flat < E, e_flat, E)
    oh = (key[:, None] == jnp.arange(E + 1, dtype=jnp.int32)[None, :]
          ).astype(jnp.int32)
    cnt = jnp.sum(oh[:, :E], axis=0).astype(jnp.int32)
    blocks_per = (cnt + TM - 1) // TM
    blk_start = jnp.concatenate(
        [jnp.zeros((1,), jnp.int32), jnp.cumsum(blocks_per)[:-1]])
    nact = jnp.sum(blocks_per).astype(jnp.int32)
    rank = jnp.sum((jnp.cumsum(oh, axis=0) - oh) * oh, axis=1)
    pid = jnp.arange(T * TOPK, dtype=jnp.int32)
    dest = jnp.where(key < E,
                     TM * blk_start[jnp.minimum(key, E - 1)] + rank,
                     NPAD + pid)
    row_tok = jnp.zeros((NPAD + T * TOPK,), jnp.int32).at[dest].set(
        tok, unique_indices=True)[:NPAD]
    row_w = jnp.zeros((NPAD + T * TOPK,), _F32).at[dest].set(
        w_flat, unique_indices=True)[:NPAD]
    bidx = jnp.arange(NB, dtype=jnp.int32)
    be = jnp.sum(bidx[:, None] >= blk_start[None, :], axis=1).astype(jnp.int32) - 1
    be_last = be[jnp.maximum(nact - 1, 0)]
    be = jnp.where(bidx < nact, be, be_last)
    nact_arr = nact.reshape(1)

    tok3 = row_tok.reshape(NB, 1, TM)
    ww3 = row_w.reshape(NB, 1, TM)

    h1 = _h1_call(be, nact_arr, tok3, x, W1, W3)

    grid_spec = pltpu.PrefetchScalarGridSpec(
        num_scalar_prefetch=2,
        grid=(NB,),
        in_specs=[
            pl.BlockSpec((1, 1, TM), lambda b, be, na: (b, 0, 0)),
            pl.BlockSpec((1, 1, TM), lambda b, be, na: (b, 0, 0)),
            pl.BlockSpec((1, 1, TM, FF2), lambda b, be, na: (0, b, 0, 0)),
            pl.BlockSpec((1, 1, TM, FF2), lambda b, be, na: (1, b, 0, 0)),
            pl.BlockSpec((1, H, FF), lambda b, be, na: (be[b], 0, 0)),
        ],
        out_specs=pl.BlockSpec((T, H), lambda b, be, na: (0, 0)),
        scratch_shapes=[
            pltpu.VMEM((H, FF), _BF16),
        ],
    )
    final = pl.pallas_call(
        _out_body,
        grid_spec=grid_spec,
        out_shape=jax.ShapeDtypeStruct((T, H), _F32),
        compiler_params=pltpu.CompilerParams(
            dimension_semantics=("arbitrary",),
            vmem_limit_bytes=60 * 1024 * 1024),
    )(be, nact_arr, tok3, ww3, h1, h1, W2)

    return final.reshape(b, s, h), router_logits
